# baseline (speedup 1.0000x reference)
import functools
import jax
import jax.numpy as jnp
from jax import lax
from jax.experimental import pallas as pl
from jax.experimental.pallas import tpu as pltpu
from jax.experimental.pallas import tpu_sc as plsc

_EMBED = 64
_WINDOW = 200
_BATCH = 1024
_ROWS = _BATCH * _WINDOW
_NW = 32
_RPW = _ROWS // _NW
_CHUNK = _WINDOW
_NCHUNK = _RPW // _CHUNK
_NBUF = 4
_OUTER = _NCHUNK // _NBUF
_SCALE = 8.0


def _body(idx_hbm, table_hbm, pos_hbm, out_hbm,
          idx_v, pos_v, b0, b1, b2, b3,
          g0, g1, g2, g3, o0, o1, o2, o3):
    bufs = [b0, b1, b2, b3]
    gsems = [g0, g1, g2, g3]
    osems = [o0, o1, o2, o3]
    wid = lax.axis_index("s") * 2 + lax.axis_index("c")
    base = wid * _RPW

    pltpu.sync_copy(idx_hbm.at[pl.ds(base, _RPW)], idx_v)
    pltpu.sync_copy(pos_hbm, pos_v)

    def gather_start(c, j):
        pltpu.make_async_copy(
            table_hbm.at[idx_v.at[pl.ds(c * _CHUNK, _CHUNK)]],
            bufs[j], gsems[j]).start()

    def gather_wait(j):
        pltpu.make_async_copy(
            table_hbm.at[idx_v.at[pl.ds(0, _CHUNK)]],
            bufs[j], gsems[j]).wait()

    def scatter_start(c, j):
        pltpu.make_async_copy(
            bufs[j], out_hbm.at[pl.ds(base + c * _CHUNK, _CHUNK)],
            osems[j]).start()

    def scatter_wait(j):
        pltpu.make_async_copy(
            bufs[j], out_hbm.at[pl.ds(0, _CHUNK)], osems[j]).wait()

    gather_start(0, 0)
    gather_start(1, 1)

    def compute(j):
        buf = bufs[j]

        def row(r, carry):
            for q in range(_EMBED // 16):
                sl = (r, pl.ds(q * 16, 16))
                buf[sl] = buf[sl] * _SCALE + pos_v[sl]
            return carry

        lax.fori_loop(0, _CHUNK, row, 0)

    def outer(i, carry):
        for j in range(_NBUF):
            c = i * _NBUF + j
            gather_wait(j)
            compute(j)
            scatter_start(c, j)
            jn = (j + 2) % _NBUF
            if j < 2:
                @pl.when(i >= 1)
                def _():
                    scatter_wait(jn)
                gather_start(c + 2, jn)
            else:
                @pl.when(i < _OUTER - 1)
                def _():
                    scatter_wait(jn)
                    gather_start(c + 2, jn)
        return carry

    lax.fori_loop(0, _OUTER, outer, 0)
    for j in range(_NBUF):
        scatter_wait(j)


_sc_call = functools.partial(
    pl.kernel,
    mesh=plsc.VectorSubcoreMesh(core_axis_name="c", subcore_axis_name="s"),
    out_type=jax.ShapeDtypeStruct((_ROWS, _EMBED), jnp.float32),
    scratch_types=[
        pltpu.VMEM((_RPW,), jnp.int32),
        pltpu.VMEM((_WINDOW, _EMBED), jnp.float32),
        pltpu.VMEM((_CHUNK, _EMBED), jnp.float32),
        pltpu.VMEM((_CHUNK, _EMBED), jnp.float32),
        pltpu.VMEM((_CHUNK, _EMBED), jnp.float32),
        pltpu.VMEM((_CHUNK, _EMBED), jnp.float32),
        pltpu.SemaphoreType.DMA,
        pltpu.SemaphoreType.DMA,
        pltpu.SemaphoreType.DMA,
        pltpu.SemaphoreType.DMA,
        pltpu.SemaphoreType.DMA,
        pltpu.SemaphoreType.DMA,
        pltpu.SemaphoreType.DMA,
        pltpu.SemaphoreType.DMA,
    ],
    compiler_params=pltpu.CompilerParams(use_tc_tiling_on_sc=False),
)(_body)


def kernel(x, table, pos_enc):
    x_flat = jnp.reshape(x, (_ROWS,)).astype(jnp.int32)
    out = _sc_call(x_flat, table, pos_enc)
    return jnp.reshape(out, (_BATCH, _WINDOW, _EMBED))


# speedup vs baseline: 1.0015x; 1.0015x over previous
import functools
import jax
import jax.numpy as jnp
from jax import lax
from jax.experimental import pallas as pl
from jax.experimental.pallas import tpu as pltpu
from jax.experimental.pallas import tpu_sc as plsc

_EMBED = 64
_WINDOW = 200
_BATCH = 1024
_NW = 32
_SPW = _BATCH // _NW
_CHUNK = _WINDOW
_NBUF = 4
_OUTER = _SPW // _NBUF
_SCALE = 8.0


def _body(idx_hbm, table_hbm, pos_hbm, out_hbm,
          idx_v, pos_v, b0, b1, b2, b3,
          g0, g1, g2, g3, o0, o1, o2, o3):
    bufs = [b0, b1, b2, b3]
    gsems = [g0, g1, g2, g3]
    osems = [o0, o1, o2, o3]
    wid = lax.axis_index("s") * 2 + lax.axis_index("c")
    seq0 = wid * _SPW

    pltpu.sync_copy(idx_hbm.at[pl.ds(seq0, _SPW)], idx_v)
    pltpu.sync_copy(pos_hbm, pos_v)

    def gather_start(c, j):
        pltpu.make_async_copy(
            table_hbm.at[idx_v.at[c]], bufs[j], gsems[j]).start()

    def gather_wait(j):
        pltpu.make_async_copy(
            table_hbm.at[idx_v.at[0]], bufs[j], gsems[j]).wait()

    def scatter_start(c, j):
        pltpu.make_async_copy(
            bufs[j], out_hbm.at[seq0 + c], osems[j]).start()

    def scatter_wait(j):
        pltpu.make_async_copy(
            bufs[j], out_hbm.at[0], osems[j]).wait()

    gather_start(0, 0)
    gather_start(1, 1)

    def compute(j):
        buf = bufs[j]

        def row(r, carry):
            for q in range(_EMBED // 16):
                sl = (r, pl.ds(q * 16, 16))
                buf[sl] = buf[sl] * _SCALE + pos_v[sl]
            return carry

        lax.fori_loop(0, _CHUNK, row, 0)

    def outer(i, carry):
        for j in range(_NBUF):
            c = i * _NBUF + j
            gather_wait(j)
            compute(j)
            scatter_start(c, j)
            jn = (j + 2) % _NBUF
            if j < 2:
                @pl.when(i >= 1)
                def _():
                    scatter_wait(jn)
                gather_start(c + 2, jn)
            else:
                @pl.when(i < _OUTER - 1)
                def _():
                    scatter_wait(jn)
                    gather_start(c + 2, jn)
        return carry

    lax.fori_loop(0, _OUTER, outer, 0)
    for j in range(_NBUF):
        scatter_wait(j)


_sc_call = functools.partial(
    pl.kernel,
    mesh=plsc.VectorSubcoreMesh(core_axis_name="c", subcore_axis_name="s"),
    out_type=jax.ShapeDtypeStruct((_BATCH, _WINDOW, _EMBED), jnp.float32),
    scratch_types=[
        pltpu.VMEM((_SPW, _WINDOW), jnp.int32),
        pltpu.VMEM((_WINDOW, _EMBED), jnp.float32),
        pltpu.VMEM((_CHUNK, _EMBED), jnp.float32),
        pltpu.VMEM((_CHUNK, _EMBED), jnp.float32),
        pltpu.VMEM((_CHUNK, _EMBED), jnp.float32),
        pltpu.VMEM((_CHUNK, _EMBED), jnp.float32),
        pltpu.SemaphoreType.DMA,
        pltpu.SemaphoreType.DMA,
        pltpu.SemaphoreType.DMA,
        pltpu.SemaphoreType.DMA,
        pltpu.SemaphoreType.DMA,
        pltpu.SemaphoreType.DMA,
        pltpu.SemaphoreType.DMA,
        pltpu.SemaphoreType.DMA,
    ],
    compiler_params=pltpu.CompilerParams(use_tc_tiling_on_sc=False),
)(_body)


def kernel(x, table, pos_enc):
    return _sc_call(x.astype(jnp.int32), table, pos_enc)
